# Initial kernel scaffold; baseline (speedup 1.0000x reference)
import jax
import jax.numpy as jnp
from jax.experimental import pallas as pl


def kernel(embs, edge_index, proteins, diseases, W1, b1, W2, b2, W3, b3, rel, Wh1, bh1, Wh2, bh2):
    raise NotImplementedError("write your pallas kernel here")


# speedup vs baseline: 5.6766x; 5.6766x over previous
import functools

import jax
import jax.numpy as jnp
from jax import lax
from jax.experimental import pallas as pl
from jax.experimental.pallas import tpu as pltpu
from jax.experimental.pallas import tpu_sc as plsc

N = 10000
E = 320000
D = 128
ZD = 512
HH = 64
NP = 64
ND = 64
NPD = NP + ND

NC = 2
NS = 16
NW = NC * NS
CH = 128
NR = 10240
SLAB = NR // NS
EPAD = 323584
EP2 = 4096


def _segsum_sc(n_tbl_rows: int, epad: int):
    nchunks = epad // (NW * CH)
    mesh = plsc.VectorSubcoreMesh(core_axis_name="c", subcore_axis_name="s")

    @functools.partial(
        pl.kernel,
        out_type=jax.ShapeDtypeStruct((NC, NR, D), jnp.float32),
        mesh=mesh,
        scratch_types=[
            pltpu.VMEM((CH,), jnp.int32),
            pltpu.VMEM((CH,), jnp.int32),
            pltpu.VMEM((CH, D), jnp.float32),
            pltpu.VMEM_SHARED((NR, D), jnp.float32),
            pltpu.SemaphoreType.DMA,
        ],
    )
    def k(tbl, gidx, sidx, zeros, out, gi, si, rows, acc, sem):
        c = lax.axis_index("c")
        s = lax.axis_index("s")
        wid = s * NC + c
        pltpu.sync_copy(zeros.at[pl.ds(s * SLAB, SLAB)],
                        acc.at[pl.ds(s * SLAB, SLAB)])
        plsc.subcore_barrier()
        base = wid * (nchunks * CH)

        @pl.loop(0, nchunks)
        def _(i):
            off = base + i * CH
            pltpu.sync_copy(gidx.at[pl.ds(off, CH)], gi)
            pltpu.sync_copy(sidx.at[pl.ds(off, CH)], si)
            pltpu.async_copy(tbl.at[gi], rows, sem).wait()
            pltpu.sync_copy(rows, acc.at[si], add=True)

        plsc.subcore_barrier()
        pltpu.sync_copy(acc.at[pl.ds(s * SLAB, SLAB)],
                        out.at[c, pl.ds(s * SLAB, SLAB)])

    return k


def _gather4_sc(rows0: int, rows3: int):
    mesh = plsc.VectorSubcoreMesh(core_axis_name="c", subcore_axis_name="s")
    blk = NPD // 16

    @functools.partial(
        pl.kernel,
        out_type=jax.ShapeDtypeStruct((4, NPD, D), jnp.float32),
        mesh=mesh,
        scratch_types=[
            pltpu.VMEM((blk,), jnp.int32),
            pltpu.VMEM((blk, D), jnp.float32),
            pltpu.SemaphoreType.DMA,
        ],
    )
    def k(t0, t1, t2, t3, idx, out, idxv, rows, sem):
        c = lax.axis_index("c")
        s = lax.axis_index("s")
        wid = s * NC + c

        @pl.when(wid < 16)
        def _():
            pltpu.sync_copy(idx.at[pl.ds(wid * blk, blk)], idxv)
            for t, tbl in enumerate((t0, t1, t2, t3)):
                pltpu.async_copy(tbl.at[idxv], rows, sem).wait()
                pltpu.sync_copy(rows, out.at[t, pl.ds(wid * blk, blk)])

    return k


_BLK = 1024
_NBLK = NR // _BLK


def _fwd_body(p0, p1, w, b, out):
    s = jnp.dot(p0[...] + p1[...], w[...], preferred_element_type=jnp.float32)
    out[...] = jnp.maximum(s + b[...], 0.0)


def _fwd_tc(p0, p1, w, bvec):
    return pl.pallas_call(
        _fwd_body,
        grid=(_NBLK,),
        in_specs=[
            pl.BlockSpec((_BLK, D), lambda i: (i, 0)),
            pl.BlockSpec((_BLK, D), lambda i: (i, 0)),
            pl.BlockSpec((D, D), lambda i: (0, 0)),
            pl.BlockSpec((1, D), lambda i: (0, 0)),
        ],
        out_specs=pl.BlockSpec((_BLK, D), lambda i: (i, 0)),
        out_shape=jax.ShapeDtypeStruct((NR, D), jnp.float32),
    )(p0, p1, w, bvec)


def _bwd_body(p0, p1, m, w, g_out, q_out):
    g = p0[...] + p1[...]
    g_out[...] = g
    ds = jnp.where(m[...] > 0.0, g, 0.0)
    q_out[...] = lax.dot_general(ds, w[...], (((1,), (1,)), ((), ())),
                                 preferred_element_type=jnp.float32)


def _bwd_tc(p0, p1, mask_src, w):
    return pl.pallas_call(
        _bwd_body,
        grid=(_NBLK,),
        in_specs=[
            pl.BlockSpec((_BLK, D), lambda i: (i, 0)),
            pl.BlockSpec((_BLK, D), lambda i: (i, 0)),
            pl.BlockSpec((_BLK, D), lambda i: (i, 0)),
            pl.BlockSpec((D, D), lambda i: (0, 0)),
        ],
        out_specs=[
            pl.BlockSpec((_BLK, D), lambda i: (i, 0)),
            pl.BlockSpec((_BLK, D), lambda i: (i, 0)),
        ],
        out_shape=[
            jax.ShapeDtypeStruct((NR, D), jnp.float32),
            jax.ShapeDtypeStruct((NR, D), jnp.float32),
        ],
    )(p0, p1, mask_src, w)


def _add_body(p0, p1, out):
    out[...] = p0[...] + p1[...]


def _add_tc(p0, p1):
    return pl.pallas_call(
        _add_body,
        grid=(_NBLK,),
        in_specs=[
            pl.BlockSpec((_BLK, D), lambda i: (i, 0)),
            pl.BlockSpec((_BLK, D), lambda i: (i, 0)),
        ],
        out_specs=pl.BlockSpec((_BLK, D), lambda i: (i, 0)),
        out_shape=jax.ShapeDtypeStruct((NR, D), jnp.float32),
    )(p0, p1)


def _decoder_body(zr, rel, wh1, wh2, bh1, bh2, w3,
                  probas_out, dz_out, q3_out):
    Z = zr[...]
    relv = rel[...]
    Zp = Z[:NP]
    Zd = Z[NP:]
    hi = lax.Precision.HIGHEST
    Dmat = jnp.dot(Zp * relv, Zd.T, preferred_element_type=jnp.float32,
                   precision=hi)
    dmin = jnp.min(Dmat)
    dmax = jnp.max(Dmat)
    dmean = jnp.mean(Dmat)
    ep = jnp.mean(Zp, axis=0, keepdims=True)
    ed = jnp.mean(Zd, axis=0, keepdims=True)
    W1h = wh1[...]
    z1 = (jnp.dot(ep, W1h[:ZD], preferred_element_type=jnp.float32,
                  precision=hi)
          + jnp.dot(ed, W1h[ZD:], preferred_element_type=jnp.float32,
                    precision=hi)
          + bh1[...])
    w2 = wh2[...]
    head = jnp.sum(z1 * w2[:, :HH])
    w_min = w2[0, HH]
    w_mean = w2[0, HH + 1]
    w_max = w2[0, HH + 2]
    probas_out[...] = (head + dmin * w_min + dmean * w_mean + dmax * w_max
                       + bh2[...])

    dz1 = w2[:, :HH]
    du = lax.dot_general(dz1, W1h, (((1,), (1,)), ((), ())),
                         preferred_element_type=jnp.float32,
                         precision=hi)
    dep = du[:, :ZD]
    ded = du[:, ZD:]
    eqmin = jnp.where(Dmat == dmin, 1.0, 0.0)
    eqmax = jnp.where(Dmat == dmax, 1.0, 0.0)
    dD = (w_mean / (NP * ND)
          + w_min * eqmin / jnp.sum(eqmin)
          + w_max * eqmax / jnp.sum(eqmax))
    dZp = (jnp.dot(dD, Zd, preferred_element_type=jnp.float32, precision=hi)
           * relv + dep / NP)
    dZd = (lax.dot_general(dD, Zp, (((0,), (0,)), ((), ())),
                           preferred_element_type=jnp.float32, precision=hi)
           * relv + ded / ND)
    DZ = jnp.concatenate([dZp, dZd], axis=0)
    dz_out[...] = DZ
    ds3 = jnp.where(Z[:, :D] > 0.0, DZ[:, :D], 0.0)
    q3_out[...] = lax.dot_general(ds3, w3[...], (((1,), (1,)), ((), ())),
                                  preferred_element_type=jnp.float32,
                                  precision=hi)


def _decoder_tc(zrows, rel, wh1, wh2, bh1, bh2, w3):
    return pl.pallas_call(
        _decoder_body,
        out_shape=[
            jax.ShapeDtypeStruct((1, 1), jnp.float32),
            jax.ShapeDtypeStruct((NPD, ZD), jnp.float32),
            jax.ShapeDtypeStruct((NPD, D), jnp.float32),
        ],
    )(zrows, rel, wh1, wh2, bh1, bh2, w3)


def kernel(embs, edge_index, proteins, diseases, W1, b1, W2, b2, W3, b3,
           rel, Wh1, bh1, Wh2, bh2):
    src = edge_index[0]
    dst = edge_index[1]
    f32 = jnp.float32

    npadf = EPAD - E
    ar_f = jnp.arange(npadf, dtype=jnp.int32)
    pg_f = ar_f % 64
    ps_f = N + (ar_f % 192)
    gi_f = jnp.concatenate([src, pg_f])
    si_f = jnp.concatenate([dst, ps_f])

    zeros = jnp.zeros((NR, D), dtype=f32)
    b1r = b1.reshape(1, D)
    b2r = b2.reshape(1, D)
    b3r = b3.reshape(1, D)

    seg_e = _segsum_sc(N, EPAD)
    seg_r = _segsum_sc(NR, EPAD)
    seg_t = _segsum_sc(NR + NPD, EPAD)
    seg_s = _segsum_sc(NPD, EP2)

    p = seg_e(embs, gi_f, si_f, zeros)
    a = _fwd_tc(p[0], p[1], W1, b1r)
    p = seg_r(a, gi_f, si_f, zeros)
    b = _fwd_tc(p[0], p[1], W2, b2r)
    p = seg_r(b, gi_f, si_f, zeros)
    c = _fwd_tc(p[0], p[1], W3, b3r)

    idx128 = jnp.concatenate([proteins, diseases])
    zr4 = _gather4_sc(NR, N)(c, b, a, embs, idx128)
    zrows = zr4.transpose(1, 0, 2).reshape(NPD, ZD)
    probas, DZ, q3_rows = _decoder_tc(
        zrows, rel.reshape(1, ZD), Wh1, Wh2.reshape(1, HH + 3),
        bh1.reshape(1, HH), bh2.reshape(1, 1), W3)
    dbs_rows = DZ[:, D:2 * D]
    das_rows = DZ[:, 2 * D:3 * D]
    des_rows = DZ[:, 3 * D:]

    ar_d = jnp.arange(EP2 - NPD, dtype=jnp.int32)
    gi_d = jnp.concatenate([jnp.arange(NPD, dtype=jnp.int32), ar_d % 64])
    si_d = jnp.concatenate([idx128, N + (ar_d % 192)])
    p = seg_s(q3_rows, gi_d, si_d, zeros)
    q3 = _add_tc(p[0], p[1])

    npadb = EPAD - E - NPD
    ar_b = jnp.arange(npadb, dtype=jnp.int32)
    gi_b = jnp.concatenate([dst, NR + jnp.arange(NPD, dtype=jnp.int32),
                            ar_b % 64])
    si_b = jnp.concatenate([src, idx128, N + (ar_b % 192)])

    p = seg_t(jnp.concatenate([q3, dbs_rows], axis=0), gi_b, si_b, zeros)
    g1, q2 = _bwd_tc(p[0], p[1], b, W2)
    p = seg_t(jnp.concatenate([q2, das_rows], axis=0), gi_b, si_b, zeros)
    g2, q1 = _bwd_tc(p[0], p[1], a, W1)
    p = seg_t(jnp.concatenate([q1, des_rows], axis=0), gi_b, si_b, zeros)
    g3 = _add_tc(p[0], p[1])

    return (probas, g3[:N], g2[:N], g1[:N])


# speedup vs baseline: 10.8807x; 1.9168x over previous
import functools

import jax
import jax.numpy as jnp
from jax import lax
from jax.experimental import pallas as pl
from jax.experimental.pallas import tpu as pltpu
from jax.experimental.pallas import tpu_sc as plsc

N = 10000
E = 320000
D = 128
ZD = 512
HH = 64
NP = 64
ND = 64
NPD = NP + ND

NC = 2
NS = 16
NW = NC * NS
CH = 128
NR = 10240
SLAB = NR // NS
EPAD = 327680
EP2 = 4096


def _segsum_sc(n_tbl_rows: int, epad: int):
    nchunks = epad // (NW * CH)
    nhalf = nchunks // 2 if nchunks > 1 else 1
    assert nchunks == 1 or (nchunks % 2 == 0 and nhalf % 2 == 0)
    mesh = plsc.VectorSubcoreMesh(core_axis_name="c", subcore_axis_name="s")

    @functools.partial(
        pl.kernel,
        out_type=jax.ShapeDtypeStruct((NC, NR, D), jnp.float32),
        mesh=mesh,
        scratch_types=[
            pltpu.VMEM((nhalf, CH), jnp.int32),
            pltpu.VMEM((nhalf, CH), jnp.int32),
            pltpu.VMEM((CH, D), jnp.float32),
            pltpu.VMEM((CH, D), jnp.float32),
            pltpu.VMEM_SHARED((NR, D), jnp.float32),
            pltpu.SemaphoreType.DMA,
            pltpu.SemaphoreType.DMA,
        ],
    )
    def k(tbl, gidx, sidx, zeros, out, gbuf, sbuf, rowsA, rowsB, acc,
          semA, semB):
        c = lax.axis_index("c")
        s = lax.axis_index("s")
        wid = s * NC + c
        pltpu.sync_copy(zeros.at[pl.ds(s * SLAB, SLAB)],
                        acc.at[pl.ds(s * SLAB, SLAB)])
        plsc.subcore_barrier()
        row0 = wid * nchunks

        def issue(j, buf, sem):
            pltpu.async_copy(tbl.at[gbuf.at[j]], buf, sem)

        def drain(buf, sem):
            pltpu.make_async_copy(tbl.at[gbuf.at[0]], buf, sem).wait()

        def scat(j, buf):
            pltpu.sync_copy(buf, acc.at[sbuf.at[j]], add=True)

        if nchunks == 1:
            pltpu.sync_copy(gidx.at[pl.ds(row0, 1)], gbuf)
            pltpu.sync_copy(sidx.at[pl.ds(row0, 1)], sbuf)
            issue(0, rowsA, semA)
            drain(rowsA, semA)
            scat(0, rowsA)
        else:
            for half in range(2):
                r0 = row0 + half * nhalf
                pltpu.sync_copy(gidx.at[pl.ds(r0, nhalf)], gbuf)
                pltpu.sync_copy(sidx.at[pl.ds(r0, nhalf)], sbuf)
                issue(0, rowsA, semA)

                @pl.loop(0, nhalf // 2)
                def _(it):
                    j = it * 2
                    issue(j + 1, rowsB, semB)
                    drain(rowsA, semA)
                    scat(j, rowsA)

                    @pl.when(j + 2 < nhalf)
                    def _():
                        issue(j + 2, rowsA, semA)

                    drain(rowsB, semB)
                    scat(j + 1, rowsB)

        plsc.subcore_barrier()
        pltpu.sync_copy(acc.at[pl.ds(s * SLAB, SLAB)],
                        out.at[c, pl.ds(s * SLAB, SLAB)])

    return k


def _gather4_sc(rows0: int, rows3: int):
    mesh = plsc.VectorSubcoreMesh(core_axis_name="c", subcore_axis_name="s")
    blk = NPD // 16

    @functools.partial(
        pl.kernel,
        out_type=jax.ShapeDtypeStruct((4, NPD, D), jnp.float32),
        mesh=mesh,
        scratch_types=[
            pltpu.VMEM((blk,), jnp.int32),
            pltpu.VMEM((blk, D), jnp.float32),
            pltpu.SemaphoreType.DMA,
        ],
    )
    def k(t0, t1, t2, t3, idx, out, idxv, rows, sem):
        c = lax.axis_index("c")
        s = lax.axis_index("s")
        wid = s * NC + c

        @pl.when(wid < 16)
        def _():
            pltpu.sync_copy(idx.at[pl.ds(wid * blk, blk)], idxv)
            for t, tbl in enumerate((t0, t1, t2, t3)):
                pltpu.async_copy(tbl.at[idxv], rows, sem).wait()
                pltpu.sync_copy(rows, out.at[t, pl.ds(wid * blk, blk)])

    return k


_BLK = 1024
_NBLK = NR // _BLK


def _fwd_body(p0, p1, w, b, out):
    s = jnp.dot(p0[...] + p1[...], w[...], preferred_element_type=jnp.float32)
    out[...] = jnp.maximum(s + b[...], 0.0)


def _fwd_tc(p0, p1, w, bvec):
    return pl.pallas_call(
        _fwd_body,
        grid=(_NBLK,),
        in_specs=[
            pl.BlockSpec((_BLK, D), lambda i: (i, 0)),
            pl.BlockSpec((_BLK, D), lambda i: (i, 0)),
            pl.BlockSpec((D, D), lambda i: (0, 0)),
            pl.BlockSpec((1, D), lambda i: (0, 0)),
        ],
        out_specs=pl.BlockSpec((_BLK, D), lambda i: (i, 0)),
        out_shape=jax.ShapeDtypeStruct((NR, D), jnp.float32),
    )(p0, p1, w, bvec)


def _bwd_body(p0, p1, m, w, g_out, q_out):
    g = p0[...] + p1[...]
    g_out[...] = g
    ds = jnp.where(m[...] > 0.0, g, 0.0)
    q_out[...] = lax.dot_general(ds, w[...], (((1,), (1,)), ((), ())),
                                 preferred_element_type=jnp.float32)


def _bwd_tc(p0, p1, mask_src, w):
    return pl.pallas_call(
        _bwd_body,
        grid=(_NBLK,),
        in_specs=[
            pl.BlockSpec((_BLK, D), lambda i: (i, 0)),
            pl.BlockSpec((_BLK, D), lambda i: (i, 0)),
            pl.BlockSpec((_BLK, D), lambda i: (i, 0)),
            pl.BlockSpec((D, D), lambda i: (0, 0)),
        ],
        out_specs=[
            pl.BlockSpec((_BLK, D), lambda i: (i, 0)),
            pl.BlockSpec((_BLK, D), lambda i: (i, 0)),
        ],
        out_shape=[
            jax.ShapeDtypeStruct((NR, D), jnp.float32),
            jax.ShapeDtypeStruct((NR, D), jnp.float32),
        ],
    )(p0, p1, mask_src, w)


def _add_body(p0, p1, out):
    out[...] = p0[...] + p1[...]


def _add_tc(p0, p1):
    return pl.pallas_call(
        _add_body,
        grid=(_NBLK,),
        in_specs=[
            pl.BlockSpec((_BLK, D), lambda i: (i, 0)),
            pl.BlockSpec((_BLK, D), lambda i: (i, 0)),
        ],
        out_specs=pl.BlockSpec((_BLK, D), lambda i: (i, 0)),
        out_shape=jax.ShapeDtypeStruct((NR, D), jnp.float32),
    )(p0, p1)


def _decoder_body(zr, rel, wh1, wh2, bh1, bh2, w3,
                  probas_out, dz_out, q3_out):
    Z = zr[...]
    relv = rel[...]
    Zp = Z[:NP]
    Zd = Z[NP:]
    hi = lax.Precision.HIGHEST
    Dmat = jnp.dot(Zp * relv, Zd.T, preferred_element_type=jnp.float32,
                   precision=hi)
    dmin = jnp.min(Dmat)
    dmax = jnp.max(Dmat)
    dmean = jnp.mean(Dmat)
    ep = jnp.mean(Zp, axis=0, keepdims=True)
    ed = jnp.mean(Zd, axis=0, keepdims=True)
    W1h = wh1[...]
    z1 = (jnp.dot(ep, W1h[:ZD], preferred_element_type=jnp.float32,
                  precision=hi)
          + jnp.dot(ed, W1h[ZD:], preferred_element_type=jnp.float32,
                    precision=hi)
          + bh1[...])
    w2 = wh2[...]
    head = jnp.sum(z1 * w2[:, :HH])
    w_min = w2[0, HH]
    w_mean = w2[0, HH + 1]
    w_max = w2[0, HH + 2]
    probas_out[...] = (head + dmin * w_min + dmean * w_mean + dmax * w_max
                       + bh2[...])

    dz1 = w2[:, :HH]
    du = lax.dot_general(dz1, W1h, (((1,), (1,)), ((), ())),
                         preferred_element_type=jnp.float32,
                         precision=hi)
    dep = du[:, :ZD]
    ded = du[:, ZD:]
    eqmin = jnp.where(Dmat == dmin, 1.0, 0.0)
    eqmax = jnp.where(Dmat == dmax, 1.0, 0.0)
    dD = (w_mean / (NP * ND)
          + w_min * eqmin / jnp.sum(eqmin)
          + w_max * eqmax / jnp.sum(eqmax))
    dZp = (jnp.dot(dD, Zd, preferred_element_type=jnp.float32, precision=hi)
           * relv + dep / NP)
    dZd = (lax.dot_general(dD, Zp, (((0,), (0,)), ((), ())),
                           preferred_element_type=jnp.float32, precision=hi)
           * relv + ded / ND)
    DZ = jnp.concatenate([dZp, dZd], axis=0)
    dz_out[...] = DZ
    ds3 = jnp.where(Z[:, :D] > 0.0, DZ[:, :D], 0.0)
    q3_out[...] = lax.dot_general(ds3, w3[...], (((1,), (1,)), ((), ())),
                                  preferred_element_type=jnp.float32,
                                  precision=hi)


def _decoder_tc(zrows, rel, wh1, wh2, bh1, bh2, w3):
    return pl.pallas_call(
        _decoder_body,
        out_shape=[
            jax.ShapeDtypeStruct((1, 1), jnp.float32),
            jax.ShapeDtypeStruct((NPD, ZD), jnp.float32),
            jax.ShapeDtypeStruct((NPD, D), jnp.float32),
        ],
    )(zrows, rel, wh1, wh2, bh1, bh2, w3)


def kernel(embs, edge_index, proteins, diseases, W1, b1, W2, b2, W3, b3,
           rel, Wh1, bh1, Wh2, bh2):
    src = edge_index[0]
    dst = edge_index[1]
    f32 = jnp.float32

    npadf = EPAD - E
    ar_f = jnp.arange(npadf, dtype=jnp.int32)
    pg_f = ar_f % 64
    ps_f = N + (ar_f % 192)
    gi_f = jnp.concatenate([src, pg_f]).reshape(EPAD // CH, CH)
    si_f = jnp.concatenate([dst, ps_f]).reshape(EPAD // CH, CH)

    zeros = jnp.zeros((NR, D), dtype=f32)
    b1r = b1.reshape(1, D)
    b2r = b2.reshape(1, D)
    b3r = b3.reshape(1, D)

    seg_e = _segsum_sc(N, EPAD)
    seg_r = _segsum_sc(NR, EPAD)
    seg_t = _segsum_sc(NR + NPD, EPAD)
    seg_s = _segsum_sc(NPD, EP2)

    p = seg_e(embs, gi_f, si_f, zeros)
    a = _fwd_tc(p[0], p[1], W1, b1r)
    p = seg_r(a, gi_f, si_f, zeros)
    b = _fwd_tc(p[0], p[1], W2, b2r)
    p = seg_r(b, gi_f, si_f, zeros)
    c = _fwd_tc(p[0], p[1], W3, b3r)

    idx128 = jnp.concatenate([proteins, diseases])
    zr4 = _gather4_sc(NR, N)(c, b, a, embs, idx128)
    zrows = zr4.transpose(1, 0, 2).reshape(NPD, ZD)
    probas, DZ, q3_rows = _decoder_tc(
        zrows, rel.reshape(1, ZD), Wh1, Wh2.reshape(1, HH + 3),
        bh1.reshape(1, HH), bh2.reshape(1, 1), W3)
    dbs_rows = DZ[:, D:2 * D]
    das_rows = DZ[:, 2 * D:3 * D]
    des_rows = DZ[:, 3 * D:]

    ar_d = jnp.arange(EP2 - NPD, dtype=jnp.int32)
    gi_d = jnp.concatenate([jnp.arange(NPD, dtype=jnp.int32),
                            ar_d % 64]).reshape(EP2 // CH, CH)
    si_d = jnp.concatenate([idx128, N + (ar_d % 192)]).reshape(EP2 // CH, CH)
    p = seg_s(q3_rows, gi_d, si_d, zeros)
    q3 = _add_tc(p[0], p[1])

    npadb = EPAD - E - NPD
    ar_b = jnp.arange(npadb, dtype=jnp.int32)
    gi_b = jnp.concatenate([dst, NR + jnp.arange(NPD, dtype=jnp.int32),
                            ar_b % 64]).reshape(EPAD // CH, CH)
    si_b = jnp.concatenate([src, idx128,
                            N + (ar_b % 192)]).reshape(EPAD // CH, CH)

    p = seg_t(jnp.concatenate([q3, dbs_rows], axis=0), gi_b, si_b, zeros)
    g1, q2 = _bwd_tc(p[0], p[1], b, W2)
    p = seg_t(jnp.concatenate([q2, das_rows], axis=0), gi_b, si_b, zeros)
    g2, q1 = _bwd_tc(p[0], p[1], a, W1)
    p = seg_t(jnp.concatenate([q1, des_rows], axis=0), gi_b, si_b, zeros)
    g3 = _add_tc(p[0], p[1])

    return (probas, g3[:N], g2[:N], g1[:N])


# speedup vs baseline: 11.8515x; 1.0892x over previous
import functools

import numpy as np

import jax
import jax.numpy as jnp
from jax import lax
from jax.experimental import pallas as pl
from jax.experimental.pallas import tpu as pltpu
from jax.experimental.pallas import tpu_sc as plsc

N = 10000
E = 320000
D = 128
ZD = 512
HH = 64
NP = 64
ND = 64
NPD = NP + ND

NC = 2
NS = 16
NW = NC * NS
CH = 128
NR = 10240
SLAB = NR // NS
EPAD = 327680
EP2 = 4096


_PGF = (np.arange(EPAD - E, dtype=np.int32) % 64)
_PSF = (N + np.arange(EPAD - E, dtype=np.int32) % 192).astype(np.int32)
_PGB = np.concatenate([
    NR + np.arange(NPD, dtype=np.int32),
    np.arange(EPAD - E - NPD, dtype=np.int32) % 64]).astype(np.int32)
_PSB = (N + np.arange(EPAD - E - NPD, dtype=np.int32) % 192).astype(np.int32)
_GI_D = np.concatenate([
    np.arange(NPD, dtype=np.int32),
    np.arange(EP2 - NPD, dtype=np.int32) % 64]).astype(
        np.int32).reshape(EP2 // CH, CH)
_PS_D = (N + np.arange(EP2 - NPD, dtype=np.int32) % 192).astype(np.int32)


def _segsum_sc(n_tbl_rows: int, epad: int):
    nchunks = epad // (NW * CH)
    nhalf = nchunks // 2 if nchunks > 1 else 1
    assert nchunks == 1 or (nchunks % 2 == 0 and nhalf % 2 == 0)
    mesh = plsc.VectorSubcoreMesh(core_axis_name="c", subcore_axis_name="s")

    @functools.partial(
        pl.kernel,
        out_type=jax.ShapeDtypeStruct((NC, NR, D), jnp.float32),
        mesh=mesh,
        scratch_types=[
            pltpu.VMEM((nhalf, CH), jnp.int32),
            pltpu.VMEM((nhalf, CH), jnp.int32),
            pltpu.VMEM((CH, D), jnp.float32),
            pltpu.VMEM((CH, D), jnp.float32),
            pltpu.VMEM_SHARED((NR, D), jnp.float32),
            pltpu.SemaphoreType.DMA,
            pltpu.SemaphoreType.DMA,
        ],
    )
    def k(tbl, gidx, sidx, zeros, out, gbuf, sbuf, rowsA, rowsB, acc,
          semA, semB):
        c = lax.axis_index("c")
        s = lax.axis_index("s")
        wid = s * NC + c
        pltpu.sync_copy(zeros.at[pl.ds(s * SLAB, SLAB)],
                        acc.at[pl.ds(s * SLAB, SLAB)])
        plsc.subcore_barrier()
        row0 = wid * nchunks

        def issue(j, buf, sem):
            pltpu.async_copy(tbl.at[gbuf.at[j]], buf, sem)

        def drain(buf, sem):
            pltpu.make_async_copy(tbl.at[gbuf.at[0]], buf, sem).wait()

        def scat(j, buf):
            pltpu.sync_copy(buf, acc.at[sbuf.at[j]], add=True)

        if nchunks == 1:
            pltpu.sync_copy(gidx.at[pl.ds(row0, 1)], gbuf)
            pltpu.sync_copy(sidx.at[pl.ds(row0, 1)], sbuf)
            issue(0, rowsA, semA)
            drain(rowsA, semA)
            scat(0, rowsA)
        else:
            for half in range(2):
                r0 = row0 + half * nhalf
                pltpu.sync_copy(gidx.at[pl.ds(r0, nhalf)], gbuf)
                pltpu.sync_copy(sidx.at[pl.ds(r0, nhalf)], sbuf)
                issue(0, rowsA, semA)

                @pl.loop(0, nhalf // 2)
                def _(it):
                    j = it * 2
                    issue(j + 1, rowsB, semB)
                    drain(rowsA, semA)
                    scat(j, rowsA)

                    @pl.when(j + 2 < nhalf)
                    def _():
                        issue(j + 2, rowsA, semA)

                    drain(rowsB, semB)
                    scat(j + 1, rowsB)

        plsc.subcore_barrier()
        pltpu.sync_copy(acc.at[pl.ds(s * SLAB, SLAB)],
                        out.at[c, pl.ds(s * SLAB, SLAB)])

    return k


def _gather4_sc(rows0: int, rows3: int):
    mesh = plsc.VectorSubcoreMesh(core_axis_name="c", subcore_axis_name="s")
    blk = NPD // 16

    @functools.partial(
        pl.kernel,
        out_type=jax.ShapeDtypeStruct((4, NPD, D), jnp.float32),
        mesh=mesh,
        scratch_types=[
            pltpu.VMEM((blk,), jnp.int32),
            pltpu.VMEM((blk, D), jnp.float32),
            pltpu.SemaphoreType.DMA,
        ],
    )
    def k(t0, t1, t2, t3, idx, out, idxv, rows, sem):
        c = lax.axis_index("c")
        s = lax.axis_index("s")
        wid = s * NC + c

        @pl.when(wid < 16)
        def _():
            pltpu.sync_copy(idx.at[pl.ds(wid * blk, blk)], idxv)
            for t, tbl in enumerate((t0, t1, t2, t3)):
                pltpu.async_copy(tbl.at[idxv], rows, sem).wait()
                pltpu.sync_copy(rows, out.at[t, pl.ds(wid * blk, blk)])

    return k


def _densify_sc():
    nchunks_t = EP2 // (NS * CH)
    mesh = plsc.VectorSubcoreMesh(core_axis_name="c", subcore_axis_name="s")

    @functools.partial(
        pl.kernel,
        out_type=jax.ShapeDtypeStruct((NR + NPD, D), jnp.float32),
        mesh=mesh,
        scratch_types=[
            pltpu.VMEM((1, CH), jnp.int32),
            pltpu.VMEM((1, CH), jnp.int32),
            pltpu.VMEM((CH, D), jnp.float32),
            pltpu.VMEM_SHARED((NR, D), jnp.float32),
            pltpu.SemaphoreType.DMA,
        ],
    )
    def k(tbl, gidx, sidx, zeros, tail, out, gbuf, sbuf, rows, acc, sem):
        c = lax.axis_index("c")
        s = lax.axis_index("s")

        @pl.when(c == 0)
        def _():
            pltpu.sync_copy(zeros.at[pl.ds(s * SLAB, SLAB)],
                            acc.at[pl.ds(s * SLAB, SLAB)])
            plsc.subcore_barrier()
            for j in range(nchunks_t):
                row = s * nchunks_t + j
                pltpu.sync_copy(gidx.at[pl.ds(row, 1)], gbuf)
                pltpu.sync_copy(sidx.at[pl.ds(row, 1)], sbuf)
                pltpu.async_copy(tbl.at[gbuf.at[0]], rows, sem).wait()
                pltpu.sync_copy(rows, acc.at[sbuf.at[0]], add=True)
            plsc.subcore_barrier()
            pltpu.sync_copy(acc.at[pl.ds(s * SLAB, SLAB)],
                            out.at[pl.ds(s * SLAB, SLAB)])

        @pl.when((c == 1) & (s == 0))
        def _():
            pltpu.sync_copy(tail, rows)
            pltpu.sync_copy(rows, out.at[pl.ds(NR, NPD)])

    return k


_BLK = 1024
_NBLK = NR // _BLK


def _p_specs():
    return [
        pl.BlockSpec((1, _BLK, D), lambda i: (0, i, 0)),
        pl.BlockSpec((1, _BLK, D), lambda i: (1, i, 0)),
    ]


def _fwd_body(p0, p1, w, b, out):
    s = jnp.dot(p0[0] + p1[0], w[...], preferred_element_type=jnp.float32)
    out[...] = jnp.maximum(s + b[...], 0.0)


def _fwd_tc(p, w, bvec):
    return pl.pallas_call(
        _fwd_body,
        grid=(_NBLK,),
        in_specs=_p_specs() + [
            pl.BlockSpec((D, D), lambda i: (0, 0)),
            pl.BlockSpec((1, D), lambda i: (0, 0)),
        ],
        out_specs=pl.BlockSpec((_BLK, D), lambda i: (i, 0)),
        out_shape=jax.ShapeDtypeStruct((NR, D), jnp.float32),
    )(p, p, w, bvec)


def _bwd_body(p0, p1, m, w, rows, g_out, q_out):
    i = pl.program_id(0)
    g = p0[0] + p1[0]
    g_out[...] = g
    ds = jnp.where(m[...] > 0.0, g, 0.0)
    q_out[...] = lax.dot_general(ds, w[...], (((1,), (1,)), ((), ())),
                                 preferred_element_type=jnp.float32)

    @pl.when(i == _NBLK)
    def _():
        q_out[:NPD] = rows[...]


def _bwd_tc(p, mask_src, w, tail_rows):
    clamp = lambda i: (jnp.minimum(i, _NBLK - 1), 0)
    clamp3 = [
        pl.BlockSpec((1, _BLK, D), lambda i: (0, jnp.minimum(i, _NBLK - 1), 0)),
        pl.BlockSpec((1, _BLK, D), lambda i: (1, jnp.minimum(i, _NBLK - 1), 0)),
    ]
    return pl.pallas_call(
        _bwd_body,
        grid=(_NBLK + 1,),
        in_specs=clamp3 + [
            pl.BlockSpec((_BLK, D), clamp),
            pl.BlockSpec((D, D), lambda i: (0, 0)),
            pl.BlockSpec((NPD, D), lambda i: (0, 0)),
        ],
        out_specs=[
            pl.BlockSpec((_BLK, D), clamp),
            pl.BlockSpec((_BLK, D), lambda i: (i, 0)),
        ],
        out_shape=[
            jax.ShapeDtypeStruct((N, D), jnp.float32),
            jax.ShapeDtypeStruct((NR + NPD, D), jnp.float32),
        ],
    )(p, p, mask_src, w, tail_rows)


def _add_body(p0, p1, out):
    out[...] = p0[0] + p1[0]


def _add_tc(p):
    return pl.pallas_call(
        _add_body,
        grid=(_NBLK,),
        in_specs=_p_specs(),
        out_specs=pl.BlockSpec((_BLK, D), lambda i: (i, 0)),
        out_shape=jax.ShapeDtypeStruct((N, D), jnp.float32),
    )(p, p)


def _decoder_body(zr, rel, wh1, wh2, bh1, bh2, w3,
                  probas_out, dz_out, q3_out):
    Z = zr[...]
    relv = rel[...]
    Zp = Z[:NP]
    Zd = Z[NP:]
    hi = lax.Precision.HIGHEST
    Dmat = jnp.dot(Zp * relv, Zd.T, preferred_element_type=jnp.float32,
                   precision=hi)
    dmin = jnp.min(Dmat)
    dmax = jnp.max(Dmat)
    dmean = jnp.mean(Dmat)
    ep = jnp.mean(Zp, axis=0, keepdims=True)
    ed = jnp.mean(Zd, axis=0, keepdims=True)
    W1h = wh1[...]
    z1 = (jnp.dot(ep, W1h[:ZD], preferred_element_type=jnp.float32,
                  precision=hi)
          + jnp.dot(ed, W1h[ZD:], preferred_element_type=jnp.float32,
                    precision=hi)
          + bh1[...])
    w2 = wh2[...]
    head = jnp.sum(z1 * w2[:, :HH])
    w_min = w2[0, HH]
    w_mean = w2[0, HH + 1]
    w_max = w2[0, HH + 2]
    probas_out[...] = (head + dmin * w_min + dmean * w_mean + dmax * w_max
                       + bh2[...])

    dz1 = w2[:, :HH]
    du = lax.dot_general(dz1, W1h, (((1,), (1,)), ((), ())),
                         preferred_element_type=jnp.float32,
                         precision=hi)
    dep = du[:, :ZD]
    ded = du[:, ZD:]
    eqmin = jnp.where(Dmat == dmin, 1.0, 0.0)
    eqmax = jnp.where(Dmat == dmax, 1.0, 0.0)
    dD = (w_mean / (NP * ND)
          + w_min * eqmin / jnp.sum(eqmin)
          + w_max * eqmax / jnp.sum(eqmax))
    dZp = (jnp.dot(dD, Zd, preferred_element_type=jnp.float32, precision=hi)
           * relv + dep / NP)
    dZd = (lax.dot_general(dD, Zp, (((0,), (0,)), ((), ())),
                           preferred_element_type=jnp.float32, precision=hi)
           * relv + ded / ND)
    DZ = jnp.concatenate([dZp, dZd], axis=0)
    dz_out[...] = DZ
    ds3 = jnp.where(Z[:, :D] > 0.0, DZ[:, :D], 0.0)
    q3_out[...] = lax.dot_general(ds3, w3[...], (((1,), (1,)), ((), ())),
                                  preferred_element_type=jnp.float32,
                                  precision=hi)


def _decoder_tc(zrows, rel, wh1, wh2, bh1, bh2, w3):
    return pl.pallas_call(
        _decoder_body,
        out_shape=[
            jax.ShapeDtypeStruct((1, 1), jnp.float32),
            jax.ShapeDtypeStruct((NPD, ZD), jnp.float32),
            jax.ShapeDtypeStruct((NPD, D), jnp.float32),
        ],
    )(zrows, rel, wh1, wh2, bh1, bh2, w3)


def kernel(embs, edge_index, proteins, diseases, W1, b1, W2, b2, W3, b3,
           rel, Wh1, bh1, Wh2, bh2):
    src = edge_index[0]
    dst = edge_index[1]

    gi_f = jnp.concatenate([src, _PGF]).reshape(EPAD // CH, CH)
    si_f = jnp.concatenate([dst, _PSF]).reshape(EPAD // CH, CH)

    zeros = jnp.zeros((NR, D), dtype=jnp.float32)
    b1r = b1.reshape(1, D)
    b2r = b2.reshape(1, D)
    b3r = b3.reshape(1, D)

    seg_e = _segsum_sc(N, EPAD)
    seg_r = _segsum_sc(NR, EPAD)
    seg_t = _segsum_sc(NR + NPD, EPAD)

    p = seg_e(embs, gi_f, si_f, zeros)
    a = _fwd_tc(p, W1, b1r)
    p = seg_r(a, gi_f, si_f, zeros)
    b = _fwd_tc(p, W2, b2r)
    p = seg_r(b, gi_f, si_f, zeros)
    c = _fwd_tc(p, W3, b3r)

    idx128 = jnp.concatenate([proteins, diseases])
    zr4 = _gather4_sc(NR, N)(c, b, a, embs, idx128)
    zrows = zr4.transpose(1, 0, 2).reshape(NPD, ZD)
    probas, DZ, q3_rows = _decoder_tc(
        zrows, rel.reshape(1, ZD), Wh1, Wh2.reshape(1, HH + 3),
        bh1.reshape(1, HH), bh2.reshape(1, 1), W3)
    dbs_rows = DZ[:, D:2 * D]
    das_rows = DZ[:, 2 * D:3 * D]
    des_rows = DZ[:, 3 * D:]

    si_d = jnp.concatenate([idx128, _PS_D]).reshape(EP2 // CH, CH)
    t3 = _densify_sc()(q3_rows, _GI_D, si_d, zeros, dbs_rows)

    gi_b = jnp.concatenate([dst, _PGB]).reshape(EPAD // CH, CH)
    si_b = jnp.concatenate([src, idx128, _PSB]).reshape(EPAD // CH, CH)

    p = seg_t(t3, gi_b, si_b, zeros)
    g1, t2 = _bwd_tc(p, b, W2, das_rows)
    p = seg_t(t2, gi_b, si_b, zeros)
    g2, t1 = _bwd_tc(p, a, W1, des_rows)
    p = seg_t(t1, gi_b, si_b, zeros)
    g3 = _add_tc(p)

    return (probas, g3, g2, g1)
